# baseline (speedup 1.0000x reference)
import functools

import jax
import jax.numpy as jnp
from jax import lax
from jax.experimental import pallas as pl
from jax.experimental.pallas import tpu as pltpu
from jax.experimental.pallas import tpu_sc as plsc

B, C, N, K = 4, 64, 10000, 16
NK = N * K
NSC = 32
IW = 128
RPC = 5
NROWS = NK // IW
NCHUNKS = NROWS // RPC
CHUNK = RPC * IW

_sc_mesh = plsc.VectorSubcoreMesh(core_axis_name="c", subcore_axis_name="s")


@functools.partial(
    pl.kernel,
    mesh=_sc_mesh,
    compiler_params=pltpu.CompilerParams(
        use_tc_tiling_on_sc=False, needs_layout_passes=False
    ),
    out_type=jax.ShapeDtypeStruct((NK, 2 * C), jnp.float32),
    scratch_types=[
        pltpu.VMEM((RPC, IW), jnp.int32),
        pltpu.VMEM((CHUNK, 2 * C), jnp.float32),
        pltpu.SemaphoreType.DMA,
    ],
)
def _sc_gather(ftab_hbm, idx_hbm, gath_hbm, idx_buf, rows_buf, sem):
    wid = lax.axis_index("s") * 2 + lax.axis_index("c")

    def step(t, carry):
        chunk_id = wid + NSC * t

        @pl.when(chunk_id < NCHUNKS)
        def _():
            r0 = chunk_id * RPC
            pltpu.sync_copy(idx_hbm.at[pl.ds(r0, RPC), :], idx_buf)
            cps = [
                pltpu.async_copy(
                    ftab_hbm.at[idx_buf.at[r]],
                    rows_buf.at[pl.ds(r * IW, IW), :],
                    sem,
                )
                for r in range(RPC)
            ]
            for cp in cps:
                cp.wait()
            pltpu.sync_copy(
                rows_buf, gath_hbm.at[pl.ds(chunk_id * CHUNK, CHUNK), :]
            )

        return carry

    lax.fori_loop(0, (NCHUNKS + NSC - 1) // NSC, step, 0)


_JB = 8000


def _concat_body(rpe_ref, gath_ref, out_alias_ref, out_ref):
    del out_alias_ref
    out_ref[0] = jnp.concatenate([rpe_ref[0], gath_ref[0][:, 0:C]], axis=1)


def _concat_body_first(rpe_ref, gath_ref, out_ref):
    out_ref[0] = jnp.concatenate([rpe_ref[0], gath_ref[0][:, 0:C]], axis=1)


def _tc_concat_batch(b, rpe_b, gath_b, prev):
    out_shape = jax.ShapeDtypeStruct((B, NK, 2 * C), jnp.float32)
    in_specs = [
        pl.BlockSpec((1, _JB, C), lambda j: (0, j, 0)),
        pl.BlockSpec((1, _JB, 2 * C), lambda j: (0, j, 0)),
    ]
    out_spec = pl.BlockSpec((1, _JB, 2 * C), lambda j: (b, j, 0))
    if prev is None:
        return pl.pallas_call(
            _concat_body_first,
            grid=(NK // _JB,),
            in_specs=in_specs,
            out_specs=out_spec,
            out_shape=out_shape,
        )(rpe_b, gath_b)
    return pl.pallas_call(
        _concat_body,
        grid=(NK // _JB,),
        in_specs=in_specs + [pl.BlockSpec(memory_space=pl.ANY)],
        out_specs=out_spec,
        out_shape=out_shape,
        input_output_aliases={2: 0},
    )(rpe_b, gath_b, prev)


def kernel(relative_position_encoding, features, neighbors):
    featT = jnp.transpose(features[:, :, :, 0], (0, 2, 1))
    out = None
    for b in range(B):
        ftab_b = jnp.pad(featT[b], ((0, 0), (0, C)))
        rpe_b = jnp.transpose(
            relative_position_encoding[b], (1, 2, 0)
        ).reshape(1, NK, C)
        idx_b = neighbors[b].reshape(NROWS, IW)
        gath_b = _sc_gather(ftab_b, idx_b).reshape(1, NK, 2 * C)
        out = _tc_concat_batch(b, rpe_b, gath_b, out)
    return jnp.transpose(out.reshape(B, N, K, 2 * C), (0, 3, 1, 2))
